# baseline (speedup 1.0000x reference)
import functools

import jax
import jax.numpy as jnp
from jax import lax
from jax.experimental import pallas as pl
from jax.experimental.pallas import tpu as pltpu
from jax.experimental.pallas import tpu_sc as plsc

VOCAB = 1000000
EMB = 32
L = 20
NC, NS = 2, 16
NW = NC * NS
CP = 64
NCHUNK = 32
ROWS = CP * L
XROWS = 1280
XCOLS = 1024
OROWS = 2048
NEG = -1e30


def _compute_chunk(rows, outc, p):
  def ld(i):
    return (rows[i * CP + p, pl.ds(0, 16)],
            rows[i * CP + p, pl.ds(16, 16)])

  v0, v1 = ld(0)
  for i in range(1, L):
    w0, w1 = ld(i)
    v0 = v0 + w0
    v1 = v1 + w1
  lane = lax.iota(jnp.int32, 16)
  blo = jnp.full((16,), NEG, jnp.float32)
  bhi = blo
  for i in range(L):
    w0, w1 = ld(i)
    bi = jnp.sum(w0 * v0 + w1 * v1)
    if i < 16:
      blo = jnp.where(lane == i, bi, blo)
    else:
      bhi = jnp.where(lane == (i - 16), bi, bhi)
  m = jnp.max(jnp.maximum(blo, bhi))
  elo = jnp.exp(blo - m)
  ehi = jnp.exp(bhi - m)
  s = jnp.sum(elo + ehi)
  clo = elo / s
  chi = ehi / s
  w0, w1 = ld(0)
  a0 = w0 * clo[0]
  a1 = w1 * clo[0]
  for i in range(1, L):
    ci = clo[i] if i < 16 else chi[i - 16]
    w0, w1 = ld(i)
    a0 = a0 + w0 * ci
    a1 = a1 + w1 * ci
  pv = jnp.full((16,), p, jnp.int32)
  plsc.store_scatter(outc, [lane, pv], a0)
  plsc.store_scatter(outc, [lane + 16, pv], a1)


_NTILE = VOCAB // 8
_DTB = 40
_DCHUNKS = _NTILE // _DTB
_DROUNDS = -(-_DCHUNKS // NW)


def _sc_depad(emb):
  mesh = plsc.VectorSubcoreMesh(
      core_axis_name="c", subcore_axis_name="s",
      num_cores=NC, num_subcores=NS)

  @functools.partial(
      pl.kernel,
      out_type=jax.ShapeDtypeStruct((VOCAB // 32, 8, 128), jnp.float32),
      mesh=mesh,
      compiler_params=pltpu.CompilerParams(
          needs_layout_passes=False, use_tc_tiling_on_sc=False),
      scratch_types=[
          pltpu.VMEM((2, _DTB, 8, EMB), jnp.float32),
          pltpu.VMEM((2, _DTB // 4, 8, 128), jnp.float32),
          pltpu.SemaphoreType.DMA,
          pltpu.SemaphoreType.DMA,
          pltpu.SemaphoreType.DMA,
          pltpu.SemaphoreType.DMA,
      ],
  )
  def k(emb_hbm, out_hbm, in_v, out_v, semi0, semi1, semo0, semo1):
    wid = lax.axis_index("s") * NC + lax.axis_index("c")
    semi = (semi0, semi1)
    semo = (semo0, semo1)

    def in_copy(g, buf):
      return pltpu.make_async_copy(
          emb_hbm.at[pl.ds(g * _DTB, _DTB)], in_v.at[buf], semi[buf])

    def out_copy(g, buf):
      return pltpu.make_async_copy(
          out_v.at[buf], out_hbm.at[pl.ds(g * (_DTB // 4), _DTB // 4)],
          semo[buf])

    def compact(buf):
      def tile_body(t, carry):
        for s in range(8):
          r = t * 8 + s
          q = r // 32
          so = (r // 4) % 8
          lo = (r % 4) * 32
          out_v[buf, q, so, pl.ds(lo, 16)] = in_v[buf, t, s, pl.ds(0, 16)]
          out_v[buf, q, so, pl.ds(lo + 16, 16)] = in_v[buf, t, s,
                                                       pl.ds(16, 16)]
        return carry
      lax.fori_loop(0, _DTB, tile_body, 0, unroll=False)

    def process(j, buf):
      g = j * NW + wid

      @pl.when(g < _DCHUNKS)
      def _():
        nxt = g + NW

        @pl.when(nxt < _DCHUNKS)
        def _():
          in_copy(nxt, 1 - buf).start()

        in_copy(g, buf).wait()

        @pl.when(j >= 2)
        def _():
          out_copy(g, buf).wait()

        compact(buf)
        out_copy(g, buf).start()

    @pl.when(wid < _DCHUNKS)
    def _():
      in_copy(wid, 0).start()

    def pair_body(i, carry):
      process(2 * i, 0)
      process(2 * i + 1, 1)
      return carry

    lax.fori_loop(0, _DROUNDS // 2, pair_body, 0, unroll=False)
    out_copy(0, 0).wait()
    out_copy(0, 1).wait()

  return k(emb)


def _sc_attention_pool(emb, xphys):
  mesh = plsc.VectorSubcoreMesh(
      core_axis_name="c", subcore_axis_name="s",
      num_cores=NC, num_subcores=NS)

  @functools.partial(
      pl.kernel,
      out_type=jax.ShapeDtypeStruct((EMB, 8, 8, 8, 128), jnp.float32),
      mesh=mesh,
      compiler_params=pltpu.CompilerParams(
          needs_layout_passes=False, use_tc_tiling_on_sc=False),
      scratch_types=[
          pltpu.VMEM((2, ROWS), jnp.int32),
          pltpu.VMEM((2, ROWS, EMB), jnp.float32),
          pltpu.VMEM((2, EMB, 65), jnp.float32),
          pltpu.SemaphoreType.DMA,
          pltpu.SemaphoreType.DMA,
          pltpu.SemaphoreType.DMA,
          pltpu.SemaphoreType.DMA,
          pltpu.SemaphoreType.DMA,
          pltpu.SemaphoreType.DMA,
      ],
  )
  def k(emb_hbm, x_hbm, out_hbm, idx_v, rows_v, outc_v,
        semi0, semi1, semg0, semg1, semo0, semo1):
    wid = lax.axis_index("s") * NC + lax.axis_index("c")
    semi = (semi0, semi1)
    semg = (semg0, semg1)
    semo = (semo0, semo1)

    def decode(c):
      q = wid * 2 + (c // 16)
      h = q // 8
      bt = q % 8
      wq = (c // 2) % 8
      hb = c % 2
      return h, bt, wq, hb

    def idx_copies(c, buf):
      h, bt, wq, hb = decode(c)
      col = wq * 128 + hb * CP
      return [
          pltpu.make_async_copy(
              x_hbm.at[(h * L + l) * 8 + bt, pl.ds(col, CP)],
              idx_v.at[buf, pl.ds(l * CP, CP)],
              semi[buf],
          )
          for l in range(L)
      ]

    def gather_copies(buf):
      return [
          pltpu.make_async_copy(
              emb_hbm.at[idx_v.at[buf, pl.ds(j * 128, 128)]],
              rows_v.at[buf, pl.ds(j * 128, 128)],
              semg[buf],
          )
          for j in range(ROWS // 128)
      ]

    def out_copies(c, buf):
      h, bt, wq, hb = decode(c)
      return [
          pltpu.make_async_copy(
              outc_v.at[buf, e, pl.ds(0, CP)],
              out_hbm.at[e, h, bt, wq, pl.ds(hb * CP, CP)],
              semo[buf],
          )
          for e in range(EMB)
      ]

    def fire(copies):
      for cp in copies:
        cp.start()

    def wait(copies):
      for cp in copies:
        cp.wait()

    def process(c, buf):
      @pl.when(c + 1 < NCHUNK)
      def _():
        wait(idx_copies(c + 1, 1 - buf))
        fire(gather_copies(1 - buf))

      wait(gather_copies(buf))

      @pl.when(c + 2 < NCHUNK)
      def _():
        fire(idx_copies(c + 2, buf))

      @pl.when(c >= 2)
      def _():
        wait(out_copies(c - 2, buf))

      def body(p, carry):
        _compute_chunk(rows_v.at[buf], outc_v.at[buf], p)
        return carry

      lax.fori_loop(0, CP, body, 0, unroll=False)
      fire(out_copies(c, buf))

    fire(idx_copies(0, 0))
    wait(idx_copies(0, 0))
    fire(gather_copies(0))
    fire(idx_copies(1, 1))

    def pair_body(i, carry):
      process(2 * i, 0)
      process(2 * i + 1, 1)
      return carry

    lax.fori_loop(0, NCHUNK // 2, pair_body, 0, unroll=False)
    wait(out_copies(NCHUNK - 2, 0))
    wait(out_copies(NCHUNK - 1, 1))

  return k(emb, xphys)


def kernel(x, embeddings):
  xv = jnp.transpose(x.astype(jnp.int32), (1, 2, 4, 3, 0))
  xv = xv.reshape(1, 8, L, 8, 8, 128)
  xv = jnp.transpose(xv, (0, 1, 2, 4, 3, 5)).reshape(XROWS, XCOLS)
  emb6 = embeddings.reshape(_NTILE, 8, EMB)
  emb_compact = _sc_depad(emb6).reshape(VOCAB, EMB)
  pooled = _sc_attention_pool(emb_compact, xv)
  o = jnp.transpose(pooled, (2, 4, 0, 1, 3))
  return o.reshape(1024, EMB, 8, 8)


# speedup vs baseline: 1.4091x; 1.4091x over previous
import functools

import jax
import jax.numpy as jnp
from jax import lax
from jax.experimental import pallas as pl
from jax.experimental.pallas import tpu as pltpu
from jax.experimental.pallas import tpu_sc as plsc

VOCAB = 1000000
EMB = 32
L = 20
NC, NS = 2, 16
NW = NC * NS
CP = 64
NCHUNK = 32
ROWS = CP * L
XROWS = 1280
XCOLS = 1024
OROWS = 2048
NEG = -1e30


def _compute_chunk(rows, outc, p):
  def ld(i):
    return (rows[i * CP + p, pl.ds(0, 16)],
            rows[i * CP + p, pl.ds(16, 16)])

  v0, v1 = ld(0)
  for i in range(1, L):
    w0, w1 = ld(i)
    v0 = v0 + w0
    v1 = v1 + w1
  lane = lax.iota(jnp.int32, 16)
  blo = jnp.full((16,), NEG, jnp.float32)
  bhi = blo
  for i in range(L):
    w0, w1 = ld(i)
    bi = jnp.sum(w0 * v0 + w1 * v1)
    if i < 16:
      blo = jnp.where(lane == i, bi, blo)
    else:
      bhi = jnp.where(lane == (i - 16), bi, bhi)
  m = jnp.max(jnp.maximum(blo, bhi))
  elo = jnp.exp(blo - m)
  ehi = jnp.exp(bhi - m)
  s = jnp.sum(elo + ehi)
  clo = elo / s
  chi = ehi / s
  w0, w1 = ld(0)
  a0 = w0 * clo[0]
  a1 = w1 * clo[0]
  for i in range(1, L):
    ci = clo[i] if i < 16 else chi[i - 16]
    w0, w1 = ld(i)
    a0 = a0 + w0 * ci
    a1 = a1 + w1 * ci
  pv = jnp.full((16,), p, jnp.int32)
  plsc.store_scatter(outc, [lane, pv], a0)
  plsc.store_scatter(outc, [lane + 16, pv], a1)


_NTILE = VOCAB // 8
_DTB = 40
_DCHUNKS = _NTILE // _DTB
_DROUNDS = -(-_DCHUNKS // NW)


def _sc_depad(emb):
  mesh = plsc.VectorSubcoreMesh(
      core_axis_name="c", subcore_axis_name="s",
      num_cores=NC, num_subcores=NS)

  @functools.partial(
      pl.kernel,
      out_type=jax.ShapeDtypeStruct((VOCAB // 32, 8, 128), jnp.float32),
      mesh=mesh,
      compiler_params=pltpu.CompilerParams(
          needs_layout_passes=False, use_tc_tiling_on_sc=True),
      scratch_types=[
          pltpu.VMEM((2, _DTB, 8, EMB), jnp.float32),
          pltpu.VMEM((2, _DTB // 4, 8, 128), jnp.float32),
          pltpu.SemaphoreType.DMA,
          pltpu.SemaphoreType.DMA,
          pltpu.SemaphoreType.DMA,
          pltpu.SemaphoreType.DMA,
      ],
  )
  def k(emb_hbm, out_hbm, in_v, out_v, semi0, semi1, semo0, semo1):
    wid = lax.axis_index("s") * NC + lax.axis_index("c")
    semi = (semi0, semi1)
    semo = (semo0, semo1)

    def in_copy(g, buf):
      return pltpu.make_async_copy(
          emb_hbm.at[pl.ds(g * _DTB, _DTB)], in_v.at[buf], semi[buf])

    def out_copy(g, buf):
      return pltpu.make_async_copy(
          out_v.at[buf], out_hbm.at[pl.ds(g * (_DTB // 4), _DTB // 4)],
          semo[buf])

    def compact(buf):
      def tile_body(t, carry):
        for s in range(8):
          r = t * 8 + s
          q = r // 32
          so = (r // 4) % 8
          lo = (r % 4) * 32
          out_v[buf, q, so, pl.ds(lo, 16)] = in_v[buf, t, s, pl.ds(0, 16)]
          out_v[buf, q, so, pl.ds(lo + 16, 16)] = in_v[buf, t, s,
                                                       pl.ds(16, 16)]
        return carry
      lax.fori_loop(0, _DTB, tile_body, 0, unroll=False)

    def process(j, buf):
      g = j * NW + wid

      @pl.when(g < _DCHUNKS)
      def _():
        nxt = g + NW

        @pl.when(nxt < _DCHUNKS)
        def _():
          in_copy(nxt, 1 - buf).start()

        in_copy(g, buf).wait()

        @pl.when(j >= 2)
        def _():
          out_copy(g, buf).wait()

        compact(buf)
        out_copy(g, buf).start()

    @pl.when(wid < _DCHUNKS)
    def _():
      in_copy(wid, 0).start()

    def pair_body(i, carry):
      process(2 * i, 0)
      process(2 * i + 1, 1)
      return carry

    lax.fori_loop(0, _DROUNDS // 2, pair_body, 0, unroll=False)
    out_copy(0, 0).wait()
    out_copy(0, 1).wait()

  return k(emb)


def _sc_attention_pool(emb, xphys):
  mesh = plsc.VectorSubcoreMesh(
      core_axis_name="c", subcore_axis_name="s",
      num_cores=NC, num_subcores=NS)

  @functools.partial(
      pl.kernel,
      out_type=jax.ShapeDtypeStruct((EMB, 8, 8, 8, 128), jnp.float32),
      mesh=mesh,
      compiler_params=pltpu.CompilerParams(
          needs_layout_passes=False, use_tc_tiling_on_sc=False),
      scratch_types=[
          pltpu.VMEM((2, ROWS), jnp.int32),
          pltpu.VMEM((2, ROWS, EMB), jnp.float32),
          pltpu.VMEM((2, EMB, 65), jnp.float32),
          pltpu.SemaphoreType.DMA,
          pltpu.SemaphoreType.DMA,
          pltpu.SemaphoreType.DMA,
          pltpu.SemaphoreType.DMA,
          pltpu.SemaphoreType.DMA,
          pltpu.SemaphoreType.DMA,
      ],
  )
  def k(emb_hbm, x_hbm, out_hbm, idx_v, rows_v, outc_v,
        semi0, semi1, semg0, semg1, semo0, semo1):
    wid = lax.axis_index("s") * NC + lax.axis_index("c")
    semi = (semi0, semi1)
    semg = (semg0, semg1)
    semo = (semo0, semo1)

    def decode(c):
      q = wid * 2 + (c // 16)
      h = q // 8
      bt = q % 8
      wq = (c // 2) % 8
      hb = c % 2
      return h, bt, wq, hb

    def idx_copies(c, buf):
      h, bt, wq, hb = decode(c)
      col = wq * 128 + hb * CP
      return [
          pltpu.make_async_copy(
              x_hbm.at[(h * L + l) * 8 + bt, pl.ds(col, CP)],
              idx_v.at[buf, pl.ds(l * CP, CP)],
              semi[buf],
          )
          for l in range(L)
      ]

    def gather_copies(buf):
      return [
          pltpu.make_async_copy(
              emb_hbm.at[idx_v.at[buf, pl.ds(j * 128, 128)]],
              rows_v.at[buf, pl.ds(j * 128, 128)],
              semg[buf],
          )
          for j in range(ROWS // 128)
      ]

    def out_copies(c, buf):
      h, bt, wq, hb = decode(c)
      return [
          pltpu.make_async_copy(
              outc_v.at[buf, e, pl.ds(0, CP)],
              out_hbm.at[e, h, bt, wq, pl.ds(hb * CP, CP)],
              semo[buf],
          )
          for e in range(EMB)
      ]

    def fire(copies):
      for cp in copies:
        cp.start()

    def wait(copies):
      for cp in copies:
        cp.wait()

    def process(c, buf):
      @pl.when(c + 1 < NCHUNK)
      def _():
        wait(idx_copies(c + 1, 1 - buf))
        fire(gather_copies(1 - buf))

      wait(gather_copies(buf))

      @pl.when(c + 2 < NCHUNK)
      def _():
        fire(idx_copies(c + 2, buf))

      @pl.when(c >= 2)
      def _():
        wait(out_copies(c - 2, buf))

      def body(p, carry):
        _compute_chunk(rows_v.at[buf], outc_v.at[buf], p)
        return carry

      lax.fori_loop(0, CP, body, 0, unroll=False)
      fire(out_copies(c, buf))

    fire(idx_copies(0, 0))
    wait(idx_copies(0, 0))
    fire(gather_copies(0))
    fire(idx_copies(1, 1))

    def pair_body(i, carry):
      process(2 * i, 0)
      process(2 * i + 1, 1)
      return carry

    lax.fori_loop(0, NCHUNK // 2, pair_body, 0, unroll=False)
    wait(out_copies(NCHUNK - 2, 0))
    wait(out_copies(NCHUNK - 1, 1))

  return k(emb, xphys)


def kernel(x, embeddings):
  xv = jnp.transpose(x.astype(jnp.int32), (1, 2, 4, 3, 0))
  xv = xv.reshape(1, 8, L, 8, 8, 128)
  xv = jnp.transpose(xv, (0, 1, 2, 4, 3, 5)).reshape(XROWS, XCOLS)
  emb6 = embeddings.reshape(_NTILE, 8, EMB)
  emb_compact = _sc_depad(emb6).reshape(VOCAB, EMB)
  pooled = _sc_attention_pool(emb_compact, xv)
  o = jnp.transpose(pooled, (2, 4, 0, 1, 3))
  return o.reshape(1024, EMB, 8, 8)
